# baseline (speedup 1.0000x reference)
import functools
import math

import jax
import jax.numpy as jnp
from jax import lax
from jax.experimental import pallas as pl
from jax.experimental.pallas import tpu as pltpu
from jax.experimental.pallas import tpu_sc as plsc

NHID = 128
RATIO = 0.2
NC, NS, NW = 2, 16, 32
EC = 128


def _rps(n):
    r = -(-(n + 1) // NS)
    return -(-r // 8) * 8


@functools.partial(jax.jit, static_argnums=(4,))
def _sc_aggregate(h, src, dst, mask, n):
    e_pad = src.shape[0]
    chunks = e_pad // (NW * EC)
    rps = _rps(n)
    n_pad = rps * NS
    dummy = n
    mesh = plsc.VectorSubcoreMesh(core_axis_name="c", subcore_axis_name="s")

    @functools.partial(
        pl.kernel, mesh=mesh,
        out_type=[
            jax.ShapeDtypeStruct((NC, n_pad, NHID), jnp.float32),
            jax.ShapeDtypeStruct((NC * n_pad,), jnp.float32),
        ],
        scratch_types=[
            pltpu.VMEM((EC,), jnp.int32),
            pltpu.VMEM((EC,), jnp.int32),
            pltpu.VMEM((EC,), jnp.float32),
            pltpu.VMEM((EC, NHID), jnp.float32),
            pltpu.VMEM((rps,), jnp.float32),
            pltpu.VMEM_SHARED((n_pad, NHID), jnp.float32),
            pltpu.VMEM_SHARED((n_pad,), jnp.float32),
            pltpu.SemaphoreType.DMA,
        ],
    )
    def agg(h_hbm, src_hbm, dst_hbm, mask_hbm,
            s_out, cnt_out, srcv, dstv, maskv, rows, cvec, s_sh, cnt_sh, sem):
        cid = lax.axis_index("c")
        sid = lax.axis_index("s")
        wid = sid * NC + cid
        r0 = sid * rps

        def zrow(r, _):
            for j in range(NHID // 16):
                rows[r, pl.ds(j * 16, 16)] = jnp.zeros((16,), jnp.float32)
            return ()
        lax.fori_loop(0, EC, zrow, ())
        for j in range(rps // 16):
            cvec[pl.ds(j * 16, 16)] = jnp.zeros((16,), jnp.float32)
        off = 0
        while off < rps:
            c = min(EC, rps - off)
            pltpu.sync_copy(rows.at[pl.ds(0, c)],
                            s_sh.at[pl.ds(r0 + off, c)])
            off += c
        pltpu.sync_copy(cvec, cnt_sh.at[pl.ds(r0, rps)])
        plsc.subcore_barrier()

        def body(i, _):
            off = (wid * chunks + i) * EC
            pltpu.sync_copy(src_hbm.at[pl.ds(off, EC)], srcv)
            pltpu.sync_copy(dst_hbm.at[pl.ds(off, EC)], dstv)
            pltpu.sync_copy(mask_hbm.at[pl.ds(off, EC)], maskv)
            for j in range(EC // 16):
                sl = pl.ds(j * 16, 16)
                m = maskv[sl]
                d = dstv[sl]
                dstv[sl] = jnp.where(m > 0.0, d, dummy)
            pltpu.async_copy(h_hbm.at[srcv], rows, sem).wait()
            pltpu.sync_copy(rows, s_sh.at[dstv], add=True)
            pltpu.sync_copy(maskv, cnt_sh.at[dstv], add=True)
            return ()

        lax.fori_loop(0, chunks, body, ())
        plsc.subcore_barrier()
        off = 0
        while off < rps:
            c = min(EC, rps - off)
            pltpu.sync_copy(s_sh.at[pl.ds(r0 + off, c)], rows.at[pl.ds(0, c)])
            pltpu.sync_copy(rows.at[pl.ds(0, c)],
                            s_out.at[cid, pl.ds(r0 + off, c)])
            off += c
        pltpu.sync_copy(cnt_sh.at[pl.ds(r0, rps)], cvec)
        pltpu.sync_copy(cvec, cnt_out.at[pl.ds(cid * n_pad + r0, rps)])

    s, cnt = agg(h, src, dst, mask)
    return s[:, :n], cnt.reshape(NC, n_pad)[:, :n]


def _dense_body(s_ref, cnt_ref, h_ref, wl_ref, bl_ref, wr_ref, pw_ref, pb_ref,
                h1_ref, a_ref, dinv_ref, base_ref):
    s = s_ref[0] + s_ref[1]
    cnt = cnt_ref[0] + cnt_ref[1]
    h = h_ref[...]
    mean = s / jnp.maximum(cnt, 1.0)
    h1 = jnp.dot(mean, wl_ref[...], preferred_element_type=jnp.float32)
    h1 = h1 + bl_ref[...] + jnp.dot(h, wr_ref[...],
                                    preferred_element_type=jnp.float32)
    h1 = jnp.maximum(h1, 0.0)
    h1_ref[...] = h1
    xw = jnp.dot(h1, pw_ref[...], preferred_element_type=jnp.float32)
    deg = cnt + 1.0
    dinv = jax.lax.rsqrt(deg)
    a_ref[...] = xw * dinv
    dinv_ref[...] = dinv
    base_ref[...] = xw / deg + pb_ref[...]


def _dense_stage(s2, cnt2, h, Wl, bl, Wr, pW, pb):
    n = h.shape[0]
    R = 400
    grid = (n // R,)
    row = pl.BlockSpec((R, NHID), lambda i: (i, 0))
    prow = pl.BlockSpec((2, R, NHID), lambda i: (0, i, 0))
    pcol = pl.BlockSpec((2, R, 1), lambda i: (0, i, 0))
    col = pl.BlockSpec((R, 1), lambda i: (i, 0))
    full = pl.BlockSpec((NHID, NHID), lambda i: (0, 0))
    vec = pl.BlockSpec((1, NHID), lambda i: (0, 0))
    pws = pl.BlockSpec((NHID, 1), lambda i: (0, 0))
    pbs = pl.BlockSpec((1, 1), lambda i: (0, 0))
    h1, a, dinv, base = pl.pallas_call(
        _dense_body,
        grid=grid,
        in_specs=[prow, pcol, row, full, vec, full, pws, pbs],
        out_specs=[row, col, col, col],
        out_shape=[
            jax.ShapeDtypeStruct((n, NHID), jnp.float32),
            jax.ShapeDtypeStruct((n, 1), jnp.float32),
            jax.ShapeDtypeStruct((n, 1), jnp.float32),
            jax.ShapeDtypeStruct((n, 1), jnp.float32),
        ],
    )(s2, cnt2[..., None], h, Wl, bl.reshape(1, NHID), Wr, pW,
      pb.reshape(1, 1))
    return h1, a[:, 0], dinv[:, 0], base[:, 0]


def _head_body(x1_ref, x2_ref, x3_ref, w1_ref, b1_ref, w2_ref, b2_ref,
               w3_ref, b3_ref, feats_ref, out_ref):
    def readout(ref):
        v = ref[...]
        mx = jnp.max(v, axis=0, keepdims=True)
        mn = jnp.mean(v, axis=0, keepdims=True)
        return jnp.concatenate([mx, mn], axis=1)

    z = readout(x1_ref) + readout(x2_ref) + readout(x3_ref)
    z = jnp.dot(z, w1_ref[...], preferred_element_type=jnp.float32)
    z = jnp.maximum(z + b1_ref[...], 0.0)
    f = jnp.dot(z, w2_ref[...], preferred_element_type=jnp.float32)
    f = jnp.maximum(f + b2_ref[...], 0.0)
    feats_ref[...] = f
    out_ref[...] = jnp.dot(f, w3_ref[...],
                           preferred_element_type=jnp.float32) + b3_ref[...]


def _head(xk1, xk2, xk3, w1, b1, w2, b2, w3, b3):
    ncls = w3.shape[1]
    grph = w2.shape[1]
    feats, out = pl.pallas_call(
        _head_body,
        out_shape=[
            jax.ShapeDtypeStruct((1, grph), jnp.float32),
            jax.ShapeDtypeStruct((1, ncls), jnp.float32),
        ],
    )(xk1, xk2, xk3, w1, b1.reshape(1, -1), w2, b2.reshape(1, -1), w3,
      b3.reshape(1, -1))
    return feats, out


def _level(h, src, dst, mask, Wl, bl, Wr, pW, pb, k):
    n = h.shape[0]
    s2, cnt2 = _sc_aggregate(h, src, dst, mask, n)
    h1, a, dinv, base = _dense_stage(s2, cnt2, h, Wl, bl, Wr, pW, pb)
    g = jnp.zeros((n,), jnp.float32).at[dst].add(a[src] * mask)
    score = jnp.tanh(dinv * g + base)
    top_scores, perm = jax.lax.top_k(score, k)
    xk = h1[perm] * top_scores[:, None]
    mapping = jnp.full((n,), -1, jnp.int32).at[perm].set(
        jnp.arange(k, dtype=jnp.int32))
    ns = mapping[src]
    nd = mapping[dst]
    valid = (ns >= 0) & (nd >= 0)
    new_mask = mask * valid.astype(h.dtype)
    ns = jnp.where(valid, ns, 0)
    nd = jnp.where(valid, nd, 0)
    return xk, ns, nd, new_mask


def kernel(x, edge_index, edge_attr, batch,
           conv1_Wl, conv1_bl, conv1_Wr, pool1_W, pool1_b,
           conv2_Wl, conv2_bl, conv2_Wr, pool2_W, pool2_b,
           conv3_Wl, conv3_bl, conv3_Wr, pool3_W, pool3_b,
           lin1_W, lin1_b, lin2_W, lin2_b, lin3_W, lin3_b):
    n = batch.shape[0]
    x = x[:n]
    e = edge_attr.shape[0]
    e_pad = -(-e // (NW * EC)) * (NW * EC)
    pad = e_pad - e
    src = jnp.concatenate([edge_index[0], jnp.zeros((pad,), jnp.int32)])
    dst = jnp.concatenate([edge_index[1], jnp.zeros((pad,), jnp.int32)])
    mask = jnp.concatenate([jnp.ones((e,), x.dtype),
                            jnp.zeros((pad,), x.dtype)])
    k1 = int(math.ceil(RATIO * n))
    k2 = int(math.ceil(RATIO * k1))
    k3 = int(math.ceil(RATIO * k2))
    xk1, src, dst, mask = _level(x, src, dst, mask, conv1_Wl, conv1_bl,
                                 conv1_Wr, pool1_W, pool1_b, k1)
    xk2, src, dst, mask = _level(xk1, src, dst, mask, conv2_Wl, conv2_bl,
                                 conv2_Wr, pool2_W, pool2_b, k2)
    xk3, src, dst, mask = _level(xk2, src, dst, mask, conv3_Wl, conv3_bl,
                                 conv3_Wr, pool3_W, pool3_b, k3)
    return _head(xk1, xk2, xk3, lin1_W, lin1_b, lin2_W, lin2_b,
                 lin3_W, lin3_b)


# speedup vs baseline: 1.0113x; 1.0113x over previous
import functools
import math

import jax
import jax.numpy as jnp
from jax import lax
from jax.experimental import pallas as pl
from jax.experimental.pallas import tpu as pltpu
from jax.experimental.pallas import tpu_sc as plsc

NHID = 128
RATIO = 0.2
NC, NS, NW = 2, 16, 32
EC = 128


def _rps(n):
    r = -(-(n + EC) // NS)
    return -(-r // 8) * 8


@functools.partial(jax.jit, static_argnums=(4,))
def _sc_aggregate(h, src, dst, mask, n):
    e_pad = src.shape[0]
    chunks = e_pad // (NW * EC)
    rps = _rps(n)
    n_pad = rps * NS
    dummy = n
    mesh = plsc.VectorSubcoreMesh(core_axis_name="c", subcore_axis_name="s")

    @functools.partial(
        pl.kernel, mesh=mesh,
        out_type=[
            jax.ShapeDtypeStruct((NC, n_pad, NHID), jnp.float32),
            jax.ShapeDtypeStruct((NC * n_pad,), jnp.float32),
        ],
        scratch_types=[
            pltpu.VMEM((EC,), jnp.int32),
            pltpu.VMEM((EC,), jnp.int32),
            pltpu.VMEM((EC,), jnp.float32),
            pltpu.VMEM((EC, NHID), jnp.float32),
            pltpu.VMEM((rps,), jnp.float32),
            pltpu.VMEM_SHARED((n_pad, NHID), jnp.float32),
            pltpu.VMEM_SHARED((n_pad,), jnp.float32),
            pltpu.SemaphoreType.DMA,
        ],
    )
    def agg(h_hbm, src_hbm, dst_hbm, mask_hbm,
            s_out, cnt_out, srcv, dstv, maskv, rows, cvec, s_sh, cnt_sh, sem):
        cid = lax.axis_index("c")
        sid = lax.axis_index("s")
        wid = sid * NC + cid
        r0 = sid * rps

        def zrow(r, _):
            for j in range(NHID // 16):
                rows[r, pl.ds(j * 16, 16)] = jnp.zeros((16,), jnp.float32)
            return ()
        lax.fori_loop(0, EC, zrow, ())
        for j in range(rps // 16):
            cvec[pl.ds(j * 16, 16)] = jnp.zeros((16,), jnp.float32)
        off = 0
        while off < rps:
            c = min(EC, rps - off)
            pltpu.sync_copy(rows.at[pl.ds(0, c)],
                            s_sh.at[pl.ds(r0 + off, c)])
            off += c
        pltpu.sync_copy(cvec, cnt_sh.at[pl.ds(r0, rps)])
        plsc.subcore_barrier()

        def body(i, _):
            off = (wid * chunks + i) * EC
            pltpu.sync_copy(src_hbm.at[pl.ds(off, EC)], srcv)
            pltpu.sync_copy(dst_hbm.at[pl.ds(off, EC)], dstv)
            pltpu.sync_copy(mask_hbm.at[pl.ds(off, EC)], maskv)
            lane = lax.iota(jnp.int32, 16)
            for j in range(EC // 16):
                sl = pl.ds(j * 16, 16)
                m = maskv[sl]
                d = dstv[sl]
                dstv[sl] = jnp.where(m > 0.0, d, dummy + j * 16 + lane)
            pltpu.async_copy(h_hbm.at[srcv], rows, sem).wait()
            pltpu.sync_copy(rows, s_sh.at[dstv], add=True)
            pltpu.sync_copy(maskv, cnt_sh.at[dstv], add=True)
            return ()

        lax.fori_loop(0, chunks, body, ())
        plsc.subcore_barrier()
        off = 0
        while off < rps:
            c = min(EC, rps - off)
            pltpu.sync_copy(s_sh.at[pl.ds(r0 + off, c)], rows.at[pl.ds(0, c)])
            pltpu.sync_copy(rows.at[pl.ds(0, c)],
                            s_out.at[cid, pl.ds(r0 + off, c)])
            off += c
        pltpu.sync_copy(cnt_sh.at[pl.ds(r0, rps)], cvec)
        pltpu.sync_copy(cvec, cnt_out.at[pl.ds(cid * n_pad + r0, rps)])

    s, cnt = agg(h, src, dst, mask)
    return s[:, :n], cnt.reshape(NC, n_pad)[:, :n]


def _dense_body(s_ref, cnt_ref, h_ref, wl_ref, bl_ref, wr_ref, pw_ref, pb_ref,
                h1_ref, a_ref, dinv_ref, base_ref):
    s = s_ref[0] + s_ref[1]
    cnt = cnt_ref[0] + cnt_ref[1]
    h = h_ref[...]
    mean = s / jnp.maximum(cnt, 1.0)
    h1 = jnp.dot(mean, wl_ref[...], preferred_element_type=jnp.float32)
    h1 = h1 + bl_ref[...] + jnp.dot(h, wr_ref[...],
                                    preferred_element_type=jnp.float32)
    h1 = jnp.maximum(h1, 0.0)
    h1_ref[...] = h1
    xw = jnp.dot(h1, pw_ref[...], preferred_element_type=jnp.float32)
    deg = cnt + 1.0
    dinv = jax.lax.rsqrt(deg)
    a_ref[...] = xw * dinv
    dinv_ref[...] = dinv
    base_ref[...] = xw / deg + pb_ref[...]


def _dense_stage(s2, cnt2, h, Wl, bl, Wr, pW, pb):
    n = h.shape[0]
    R = 400
    grid = (n // R,)
    row = pl.BlockSpec((R, NHID), lambda i: (i, 0))
    prow = pl.BlockSpec((2, R, NHID), lambda i: (0, i, 0))
    pcol = pl.BlockSpec((2, R, 1), lambda i: (0, i, 0))
    col = pl.BlockSpec((R, 1), lambda i: (i, 0))
    full = pl.BlockSpec((NHID, NHID), lambda i: (0, 0))
    vec = pl.BlockSpec((1, NHID), lambda i: (0, 0))
    pws = pl.BlockSpec((NHID, 1), lambda i: (0, 0))
    pbs = pl.BlockSpec((1, 1), lambda i: (0, 0))
    h1, a, dinv, base = pl.pallas_call(
        _dense_body,
        grid=grid,
        in_specs=[prow, pcol, row, full, vec, full, pws, pbs],
        out_specs=[row, col, col, col],
        out_shape=[
            jax.ShapeDtypeStruct((n, NHID), jnp.float32),
            jax.ShapeDtypeStruct((n, 1), jnp.float32),
            jax.ShapeDtypeStruct((n, 1), jnp.float32),
            jax.ShapeDtypeStruct((n, 1), jnp.float32),
        ],
    )(s2, cnt2[..., None], h, Wl, bl.reshape(1, NHID), Wr, pW,
      pb.reshape(1, 1))
    return h1, a[:, 0], dinv[:, 0], base[:, 0]


def _head_body(x1_ref, x2_ref, x3_ref, w1_ref, b1_ref, w2_ref, b2_ref,
               w3_ref, b3_ref, feats_ref, out_ref):
    def readout(ref):
        v = ref[...]
        mx = jnp.max(v, axis=0, keepdims=True)
        mn = jnp.mean(v, axis=0, keepdims=True)
        return jnp.concatenate([mx, mn], axis=1)

    z = readout(x1_ref) + readout(x2_ref) + readout(x3_ref)
    z = jnp.dot(z, w1_ref[...], preferred_element_type=jnp.float32)
    z = jnp.maximum(z + b1_ref[...], 0.0)
    f = jnp.dot(z, w2_ref[...], preferred_element_type=jnp.float32)
    f = jnp.maximum(f + b2_ref[...], 0.0)
    feats_ref[...] = f
    out_ref[...] = jnp.dot(f, w3_ref[...],
                           preferred_element_type=jnp.float32) + b3_ref[...]


def _head(xk1, xk2, xk3, w1, b1, w2, b2, w3, b3):
    ncls = w3.shape[1]
    grph = w2.shape[1]
    feats, out = pl.pallas_call(
        _head_body,
        out_shape=[
            jax.ShapeDtypeStruct((1, grph), jnp.float32),
            jax.ShapeDtypeStruct((1, ncls), jnp.float32),
        ],
    )(xk1, xk2, xk3, w1, b1.reshape(1, -1), w2, b2.reshape(1, -1), w3,
      b3.reshape(1, -1))
    return feats, out


def _level(h, src, dst, mask, Wl, bl, Wr, pW, pb, k):
    n = h.shape[0]
    s2, cnt2 = _sc_aggregate(h, src, dst, mask, n)
    h1, a, dinv, base = _dense_stage(s2, cnt2, h, Wl, bl, Wr, pW, pb)
    g = jnp.zeros((n,), jnp.float32).at[dst].add(a[src] * mask)
    score = jnp.tanh(dinv * g + base)
    top_scores, perm = jax.lax.top_k(score, k)
    xk = h1[perm] * top_scores[:, None]
    mapping = jnp.full((n,), -1, jnp.int32).at[perm].set(
        jnp.arange(k, dtype=jnp.int32))
    ns = mapping[src]
    nd = mapping[dst]
    valid = (ns >= 0) & (nd >= 0)
    new_mask = mask * valid.astype(h.dtype)
    ns = jnp.where(valid, ns, 0)
    nd = jnp.where(valid, nd, 0)
    return xk, ns, nd, new_mask


def kernel(x, edge_index, edge_attr, batch,
           conv1_Wl, conv1_bl, conv1_Wr, pool1_W, pool1_b,
           conv2_Wl, conv2_bl, conv2_Wr, pool2_W, pool2_b,
           conv3_Wl, conv3_bl, conv3_Wr, pool3_W, pool3_b,
           lin1_W, lin1_b, lin2_W, lin2_b, lin3_W, lin3_b):
    n = batch.shape[0]
    x = x[:n]
    e = edge_attr.shape[0]
    e_pad = -(-e // (NW * EC)) * (NW * EC)
    pad = e_pad - e
    src = jnp.concatenate([edge_index[0], jnp.zeros((pad,), jnp.int32)])
    dst = jnp.concatenate([edge_index[1], jnp.zeros((pad,), jnp.int32)])
    mask = jnp.concatenate([jnp.ones((e,), x.dtype),
                            jnp.zeros((pad,), x.dtype)])
    k1 = int(math.ceil(RATIO * n))
    k2 = int(math.ceil(RATIO * k1))
    k3 = int(math.ceil(RATIO * k2))
    xk1, src, dst, mask = _level(x, src, dst, mask, conv1_Wl, conv1_bl,
                                 conv1_Wr, pool1_W, pool1_b, k1)
    xk2, src, dst, mask = _level(xk1, src, dst, mask, conv2_Wl, conv2_bl,
                                 conv2_Wr, pool2_W, pool2_b, k2)
    xk3, src, dst, mask = _level(xk2, src, dst, mask, conv3_Wl, conv3_bl,
                                 conv3_Wr, pool3_W, pool3_b, k3)
    return _head(xk1, xk2, xk3, lin1_W, lin1_b, lin2_W, lin2_b,
                 lin3_W, lin3_b)


# speedup vs baseline: 7.8520x; 7.7639x over previous
import functools
import math

import jax
import jax.numpy as jnp
from jax import lax
from jax.experimental import pallas as pl
from jax.experimental.pallas import tpu as pltpu
from jax.experimental.pallas import tpu_sc as plsc

NHID = 128
RATIO = 0.2
NC, NS, NW = 2, 16, 32
EC = 128


def _rps(n):
    r = -(-(n + EC) // NS)
    return -(-r // 8) * 8


@functools.partial(jax.jit, static_argnums=(5,))
def _sc_aggregate(h, src, dst, mask, counts, n):
    e_pad = src.shape[0]
    epw = e_pad // NW
    rps = _rps(n)
    n_pad = rps * NS
    dummy = n
    mesh = plsc.VectorSubcoreMesh(core_axis_name="c", subcore_axis_name="s")

    @functools.partial(
        pl.kernel, mesh=mesh,
        out_type=[
            jax.ShapeDtypeStruct((NC, n_pad, NHID), jnp.float32),
            jax.ShapeDtypeStruct((NC * n_pad,), jnp.float32),
        ],
        scratch_types=[
            pltpu.VMEM((EC,), jnp.int32),
            pltpu.VMEM((EC,), jnp.int32),
            pltpu.VMEM((EC,), jnp.float32),
            pltpu.VMEM((EC, NHID), jnp.float32),
            pltpu.VMEM((-(-rps // 16) * 16,), jnp.float32),
            pltpu.VMEM_SHARED((n_pad, NHID), jnp.float32),
            pltpu.VMEM_SHARED((n_pad,), jnp.float32),
            pltpu.VMEM((16,), jnp.int32),
            pltpu.SemaphoreType.DMA,
        ],
    )
    def agg(h_hbm, src_hbm, dst_hbm, mask_hbm, counts_hbm,
            s_out, cnt_out, srcv, dstv, maskv, rows, cvec, s_sh, cnt_sh,
            cnts, sem):
        cid = lax.axis_index("c")
        sid = lax.axis_index("s")
        wid = sid * NC + cid
        r0 = sid * rps
        pltpu.sync_copy(counts_hbm.at[pl.ds(wid * 16, 16)], cnts)
        cv = cnts[...]
        nch = (cv[0] + (EC - 1)) // EC
        base = pl.multiple_of(cv[1], EC)

        def zrow(r, _):
            for j in range(NHID // 16):
                rows[r, pl.ds(j * 16, 16)] = jnp.zeros((16,), jnp.float32)
            return ()
        lax.fori_loop(0, EC, zrow, ())
        for j in range(-(-rps // 16)):
            cvec[pl.ds(j * 16, 16)] = jnp.zeros((16,), jnp.float32)
        off = 0
        while off < rps:
            c = min(EC, rps - off)
            pltpu.sync_copy(rows.at[pl.ds(0, c)],
                            s_sh.at[pl.ds(r0 + off, c)])
            off += c
        pltpu.sync_copy(cvec.at[pl.ds(0, rps)], cnt_sh.at[pl.ds(r0, rps)])
        plsc.subcore_barrier()

        def body(i, _):
            off = pl.multiple_of(base + i * EC, EC)
            pltpu.sync_copy(src_hbm.at[pl.ds(off, EC)], srcv)
            pltpu.sync_copy(dst_hbm.at[pl.ds(off, EC)], dstv)
            pltpu.sync_copy(mask_hbm.at[pl.ds(off, EC)], maskv)
            lane = lax.iota(jnp.int32, 16)
            for j in range(EC // 16):
                sl = pl.ds(j * 16, 16)
                m = maskv[sl]
                d = dstv[sl]
                dstv[sl] = jnp.where(m > 0.0, d, dummy + j * 16 + lane)
            pltpu.async_copy(h_hbm.at[srcv], rows, sem).wait()
            pltpu.sync_copy(rows, s_sh.at[dstv], add=True)
            pltpu.sync_copy(maskv, cnt_sh.at[dstv], add=True)
            return ()

        lax.fori_loop(0, nch, body, ())
        plsc.subcore_barrier()
        off = 0
        while off < rps:
            c = min(EC, rps - off)
            pltpu.sync_copy(s_sh.at[pl.ds(r0 + off, c)], rows.at[pl.ds(0, c)])
            pltpu.sync_copy(rows.at[pl.ds(0, c)],
                            s_out.at[cid, pl.ds(r0 + off, c)])
            off += c
        pltpu.sync_copy(cnt_sh.at[pl.ds(r0, rps)], cvec.at[pl.ds(0, rps)])
        pltpu.sync_copy(cvec.at[pl.ds(0, rps)],
                        cnt_out.at[pl.ds(cid * n_pad + r0, rps)])

    s, cnt = agg(h, src, dst, mask, counts)
    return s[:, :n], cnt.reshape(NC, n_pad)[:, :n]


@functools.partial(jax.jit, static_argnums=(5,))
def _sc_gscatter(a, src, dst, mask, counts, n):
    e_pad = src.shape[0]
    epw = e_pad // NW
    rps = _rps(n)
    n_pad = rps * NS
    dummy = n
    mesh = plsc.VectorSubcoreMesh(core_axis_name="c", subcore_axis_name="s")

    @functools.partial(
        pl.kernel, mesh=mesh,
        out_type=jax.ShapeDtypeStruct((NC * n_pad,), jnp.float32),
        scratch_types=[
            pltpu.VMEM((EC,), jnp.int32),
            pltpu.VMEM((EC,), jnp.int32),
            pltpu.VMEM((EC,), jnp.float32),
            pltpu.VMEM((EC,), jnp.float32),
            pltpu.VMEM((-(-rps // 16) * 16,), jnp.float32),
            pltpu.VMEM_SHARED((n_pad,), jnp.float32),
            pltpu.VMEM((16,), jnp.int32),
            pltpu.SemaphoreType.DMA,
        ],
    )
    def gsc(a_hbm, src_hbm, dst_hbm, mask_hbm, counts_hbm,
            g_out, srcv, dstv, maskv, valsv, cvec, g_sh, cnts, sem):
        cid = lax.axis_index("c")
        sid = lax.axis_index("s")
        wid = sid * NC + cid
        r0 = sid * rps
        pltpu.sync_copy(counts_hbm.at[pl.ds(wid * 16, 16)], cnts)
        cv = cnts[...]
        nch = (cv[0] + (EC - 1)) // EC
        base = pl.multiple_of(cv[1], EC)
        for j in range(-(-rps // 16)):
            cvec[pl.ds(j * 16, 16)] = jnp.zeros((16,), jnp.float32)
        pltpu.sync_copy(cvec.at[pl.ds(0, rps)], g_sh.at[pl.ds(r0, rps)])
        plsc.subcore_barrier()

        def body(i, _):
            off = pl.multiple_of(base + i * EC, EC)
            pltpu.sync_copy(src_hbm.at[pl.ds(off, EC)], srcv)
            pltpu.sync_copy(dst_hbm.at[pl.ds(off, EC)], dstv)
            pltpu.sync_copy(mask_hbm.at[pl.ds(off, EC)], maskv)
            lane = lax.iota(jnp.int32, 16)
            for j in range(EC // 16):
                sl = pl.ds(j * 16, 16)
                m = maskv[sl]
                d = dstv[sl]
                dstv[sl] = jnp.where(m > 0.0, d, dummy + j * 16 + lane)
            pltpu.async_copy(a_hbm.at[srcv], valsv, sem).wait()
            pltpu.sync_copy(valsv, g_sh.at[dstv], add=True)
            return ()

        lax.fori_loop(0, nch, body, ())
        plsc.subcore_barrier()
        pltpu.sync_copy(g_sh.at[pl.ds(r0, rps)], cvec.at[pl.ds(0, rps)])
        pltpu.sync_copy(cvec.at[pl.ds(0, rps)],
                        g_out.at[pl.ds(cid * n_pad + r0, rps)])

    g = gsc(a, src, dst, mask, counts)
    return g.reshape(NC, n_pad)[:, :n]


def _sc_edgemap(src, dst, mask, counts, mapping):
    e_pad = src.shape[0]
    epw = e_pad // NW
    cpw = epw // EC
    mesh = plsc.VectorSubcoreMesh(core_axis_name="c", subcore_axis_name="s")

    @functools.partial(
        pl.kernel, mesh=mesh,
        out_type=[
            jax.ShapeDtypeStruct((e_pad,), jnp.int32),
            jax.ShapeDtypeStruct((e_pad,), jnp.int32),
            jax.ShapeDtypeStruct((e_pad,), jnp.float32),
        ],
        scratch_types=[
            pltpu.VMEM((EC,), jnp.int32),
            pltpu.VMEM((EC,), jnp.int32),
            pltpu.VMEM((EC,), jnp.int32),
            pltpu.VMEM((EC,), jnp.int32),
            pltpu.VMEM((EC,), jnp.float32),
            pltpu.VMEM((EC,), jnp.float32),
            pltpu.VMEM((16,), jnp.int32),
            pltpu.SemaphoreType.DMA,
        ],
    )
    def emap(src_hbm, dst_hbm, mask_hbm, counts_hbm, map_hbm,
             ms_out, md_out, nm_out,
             msv, mdv, srcv, dstv, maskv, nmv, cnts, sem):
        cid = lax.axis_index("c")
        sid = lax.axis_index("s")
        wid = sid * NC + cid
        pltpu.sync_copy(counts_hbm.at[pl.ds(wid * 16, 16)], cnts)
        cv = cnts[...]
        nch = (cv[0] + (EC - 1)) // EC
        base = pl.multiple_of(cv[1], EC)

        def body(i, _):
            off = pl.multiple_of(base + i * EC, EC)
            pltpu.sync_copy(src_hbm.at[pl.ds(off, EC)], srcv)
            pltpu.sync_copy(dst_hbm.at[pl.ds(off, EC)], dstv)
            pltpu.sync_copy(mask_hbm.at[pl.ds(off, EC)], maskv)
            pltpu.async_copy(map_hbm.at[srcv], msv, sem).wait()
            pltpu.async_copy(map_hbm.at[dstv], mdv, sem).wait()
            for j in range(EC // 16):
                sl = pl.ds(j * 16, 16)
                ms = msv[sl]
                md = mdv[sl]
                valid = (ms >= 0) & (md >= 0) & (maskv[sl] > 0.0)
                msv[sl] = jnp.where(valid, ms, 0)
                mdv[sl] = jnp.where(valid, md, 0)
                nmv[sl] = jnp.where(valid, 1.0, 0.0)
            pltpu.sync_copy(msv, ms_out.at[pl.ds(off, EC)])
            pltpu.sync_copy(mdv, md_out.at[pl.ds(off, EC)])
            pltpu.sync_copy(nmv, nm_out.at[pl.ds(off, EC)])
            return ()

        lax.fori_loop(0, nch, body, ())

    return emap(src, dst, mask, counts, mapping)


def _compact(ms, md, nm, prev_mask):
    e_pad = ms.shape[0]
    nmi = jnp.where(prev_mask > 0, nm, 0.0).astype(jnp.int32)
    cs = jnp.cumsum(nmi)
    total = cs[-1]
    pos = jnp.where(nmi > 0, cs - 1, e_pad + 16)
    src2 = jnp.zeros((e_pad,), jnp.int32).at[pos].set(ms, mode="drop")
    dst2 = jnp.zeros((e_pad,), jnp.int32).at[pos].set(md, mode="drop")
    mask2 = (jnp.arange(e_pad) < total).astype(jnp.float32)
    w = jnp.arange(NW)
    starts = ((w * total) // NW) // EC * EC
    ends = jnp.concatenate([starts[1:], total[None]])
    cw = jnp.maximum(ends - starts, 0)
    counts2 = (jnp.zeros((NW, 16), jnp.int32)
               .at[:, 0].set(cw).at[:, 1].set(starts).reshape(-1))
    return src2, dst2, mask2, counts2


def _dense_body(s_ref, cnt_ref, h_ref, wl_ref, bl_ref, wr_ref, pw_ref, pb_ref,
                h1_ref, a_ref, dinv_ref, base_ref):
    s = s_ref[0] + s_ref[1]
    cnt = cnt_ref[0] + cnt_ref[1]
    h = h_ref[...]
    mean = s / jnp.maximum(cnt, 1.0)
    h1 = jnp.dot(mean, wl_ref[...], preferred_element_type=jnp.float32)
    h1 = h1 + bl_ref[...] + jnp.dot(h, wr_ref[...],
                                    preferred_element_type=jnp.float32)
    h1 = jnp.maximum(h1, 0.0)
    h1_ref[...] = h1
    xw = jnp.dot(h1, pw_ref[...], preferred_element_type=jnp.float32)
    deg = cnt + 1.0
    dinv = jax.lax.rsqrt(deg)
    a_ref[...] = xw * dinv
    dinv_ref[...] = dinv
    base_ref[...] = xw / deg + pb_ref[...]


def _dense_stage(s2, cnt2, h, Wl, bl, Wr, pW, pb):
    n = h.shape[0]
    R = 400
    grid = (n // R,)
    row = pl.BlockSpec((R, NHID), lambda i: (i, 0))
    prow = pl.BlockSpec((2, R, NHID), lambda i: (0, i, 0))
    pcol = pl.BlockSpec((2, R, 1), lambda i: (0, i, 0))
    col = pl.BlockSpec((R, 1), lambda i: (i, 0))
    full = pl.BlockSpec((NHID, NHID), lambda i: (0, 0))
    vec = pl.BlockSpec((1, NHID), lambda i: (0, 0))
    pws = pl.BlockSpec((NHID, 1), lambda i: (0, 0))
    pbs = pl.BlockSpec((1, 1), lambda i: (0, 0))
    h1, a, dinv, base = pl.pallas_call(
        _dense_body,
        grid=grid,
        in_specs=[prow, pcol, row, full, vec, full, pws, pbs],
        out_specs=[row, col, col, col],
        out_shape=[
            jax.ShapeDtypeStruct((n, NHID), jnp.float32),
            jax.ShapeDtypeStruct((n, 1), jnp.float32),
            jax.ShapeDtypeStruct((n, 1), jnp.float32),
            jax.ShapeDtypeStruct((n, 1), jnp.float32),
        ],
    )(s2, cnt2[..., None], h, Wl, bl.reshape(1, NHID), Wr, pW,
      pb.reshape(1, 1))
    return h1, a[:, 0], dinv[:, 0], base[:, 0]


def _head_body(x1_ref, x2_ref, x3_ref, w1_ref, b1_ref, w2_ref, b2_ref,
               w3_ref, b3_ref, feats_ref, out_ref):
    def readout(ref):
        v = ref[...]
        mx = jnp.max(v, axis=0, keepdims=True)
        mn = jnp.mean(v, axis=0, keepdims=True)
        return jnp.concatenate([mx, mn], axis=1)

    z = readout(x1_ref) + readout(x2_ref) + readout(x3_ref)
    z = jnp.dot(z, w1_ref[...], preferred_element_type=jnp.float32)
    z = jnp.maximum(z + b1_ref[...], 0.0)
    f = jnp.dot(z, w2_ref[...], preferred_element_type=jnp.float32)
    f = jnp.maximum(f + b2_ref[...], 0.0)
    feats_ref[...] = f
    out_ref[...] = jnp.dot(f, w3_ref[...],
                           preferred_element_type=jnp.float32) + b3_ref[...]


def _head(xk1, xk2, xk3, w1, b1, w2, b2, w3, b3):
    ncls = w3.shape[1]
    grph = w2.shape[1]
    feats, out = pl.pallas_call(
        _head_body,
        out_shape=[
            jax.ShapeDtypeStruct((1, grph), jnp.float32),
            jax.ShapeDtypeStruct((1, ncls), jnp.float32),
        ],
    )(xk1, xk2, xk3, w1, b1.reshape(1, -1), w2, b2.reshape(1, -1), w3,
      b3.reshape(1, -1))
    return feats, out


def _level(h, src, dst, mask, counts, Wl, bl, Wr, pW, pb, k, last=False):
    n = h.shape[0]
    s2, cnt2 = _sc_aggregate(h, src, dst, mask, counts, n)
    h1, a, dinv, base = _dense_stage(s2, cnt2, h, Wl, bl, Wr, pW, pb)
    g2 = _sc_gscatter(a, src, dst, mask, counts, n)
    score = jnp.tanh(dinv * (g2[0] + g2[1]) + base)
    top_scores, perm = jax.lax.top_k(score, k)
    xk = h1[perm] * top_scores[:, None]
    if last:
        return xk, None, None, None, None
    mapping = jnp.full((n,), -1, jnp.int32).at[perm].set(
        jnp.arange(k, dtype=jnp.int32))
    ms, md, nm = _sc_edgemap(src, dst, mask, counts, mapping)
    src2, dst2, mask2, counts2 = _compact(ms, md, nm, mask)
    return xk, src2, dst2, mask2, counts2


def kernel(x, edge_index, edge_attr, batch,
           conv1_Wl, conv1_bl, conv1_Wr, pool1_W, pool1_b,
           conv2_Wl, conv2_bl, conv2_Wr, pool2_W, pool2_b,
           conv3_Wl, conv3_bl, conv3_Wr, pool3_W, pool3_b,
           lin1_W, lin1_b, lin2_W, lin2_b, lin3_W, lin3_b):
    n = batch.shape[0]
    x = x[:n]
    e = edge_attr.shape[0]
    e_pad = -(-e // (NW * EC)) * (NW * EC)
    pad = e_pad - e
    epw = e_pad // NW
    src = jnp.concatenate([edge_index[0], jnp.zeros((pad,), jnp.int32)])
    dst = jnp.concatenate([edge_index[1], jnp.zeros((pad,), jnp.int32)])
    mask = jnp.concatenate([jnp.ones((e,), x.dtype),
                            jnp.zeros((pad,), x.dtype)])
    w = jnp.arange(NW)
    counts = (jnp.zeros((NW, 16), jnp.int32)
              .at[:, 0].set(epw).at[:, 1].set(w * epw).reshape(-1))
    k1 = int(math.ceil(RATIO * n))
    k2 = int(math.ceil(RATIO * k1))
    k3 = int(math.ceil(RATIO * k2))
    xk1, src, dst, mask, counts = _level(
        x, src, dst, mask, counts, conv1_Wl, conv1_bl, conv1_Wr,
        pool1_W, pool1_b, k1)
    xk2, src, dst, mask, counts = _level(
        xk1, src, dst, mask, counts, conv2_Wl, conv2_bl, conv2_Wr,
        pool2_W, pool2_b, k2)
    xk3, _, _, _, _ = _level(
        xk2, src, dst, mask, counts, conv3_Wl, conv3_bl, conv3_Wr,
        pool3_W, pool3_b, k3, last=True)
    return _head(xk1, xk2, xk3, lin1_W, lin1_b, lin2_W, lin2_b,
                 lin3_W, lin3_b)


# speedup vs baseline: 37.4025x; 4.7635x over previous
import functools
import math

import jax
import jax.numpy as jnp
from jax import lax
from jax.experimental import pallas as pl
from jax.experimental.pallas import tpu as pltpu
from jax.experimental.pallas import tpu_sc as plsc

NHID = 128
RATIO = 0.2
NC, NS, NW = 2, 16, 32
EC = 128


def _rps(n):
    r = -(-(n + EC) // NS)
    return -(-r // 8) * 8


@functools.partial(jax.jit, static_argnums=(5,))
def _sc_aggregate(h, src, dst, mask, counts, n):
    e_pad = src.shape[0]
    epw = e_pad // NW
    rps = _rps(n)
    n_pad = rps * NS
    dummy = n
    mesh = plsc.VectorSubcoreMesh(core_axis_name="c", subcore_axis_name="s")

    @functools.partial(
        pl.kernel, mesh=mesh,
        out_type=[
            jax.ShapeDtypeStruct((NC, n_pad, NHID), jnp.float32),
            jax.ShapeDtypeStruct((NC * n_pad,), jnp.float32),
        ],
        scratch_types=[
            pltpu.VMEM((EC,), jnp.int32),
            pltpu.VMEM((EC,), jnp.int32),
            pltpu.VMEM((EC,), jnp.float32),
            pltpu.VMEM((EC, NHID), jnp.float32),
            pltpu.VMEM((-(-rps // 16) * 16,), jnp.float32),
            pltpu.VMEM_SHARED((n_pad, NHID), jnp.float32),
            pltpu.VMEM_SHARED((n_pad,), jnp.float32),
            pltpu.VMEM((16,), jnp.int32),
            pltpu.SemaphoreType.DMA,
        ],
    )
    def agg(h_hbm, src_hbm, dst_hbm, mask_hbm, counts_hbm,
            s_out, cnt_out, srcv, dstv, maskv, rows, cvec, s_sh, cnt_sh,
            cnts, sem):
        cid = lax.axis_index("c")
        sid = lax.axis_index("s")
        wid = sid * NC + cid
        r0 = sid * rps
        pltpu.sync_copy(counts_hbm.at[pl.ds(wid * 16, 16)], cnts)
        cv = cnts[...]
        nch = (cv[0] + (EC - 1)) // EC
        base = pl.multiple_of(cv[1], EC)

        def zrow(r, _):
            for j in range(NHID // 16):
                rows[r, pl.ds(j * 16, 16)] = jnp.zeros((16,), jnp.float32)
            return ()
        lax.fori_loop(0, EC, zrow, ())
        for j in range(-(-rps // 16)):
            cvec[pl.ds(j * 16, 16)] = jnp.zeros((16,), jnp.float32)
        off = 0
        while off < rps:
            c = min(EC, rps - off)
            pltpu.sync_copy(rows.at[pl.ds(0, c)],
                            s_sh.at[pl.ds(r0 + off, c)])
            off += c
        pltpu.sync_copy(cvec.at[pl.ds(0, rps)], cnt_sh.at[pl.ds(r0, rps)])
        plsc.subcore_barrier()

        def body(i, _):
            off = pl.multiple_of(base + i * EC, EC)
            pltpu.sync_copy(src_hbm.at[pl.ds(off, EC)], srcv)
            pltpu.sync_copy(dst_hbm.at[pl.ds(off, EC)], dstv)
            pltpu.sync_copy(mask_hbm.at[pl.ds(off, EC)], maskv)
            lane = lax.iota(jnp.int32, 16)
            for j in range(EC // 16):
                sl = pl.ds(j * 16, 16)
                m = maskv[sl]
                d = dstv[sl]
                dstv[sl] = jnp.where(m > 0.0, d, dummy + j * 16 + lane)
            pltpu.async_copy(h_hbm.at[srcv], rows, sem).wait()
            pltpu.sync_copy(rows, s_sh.at[dstv], add=True)
            pltpu.sync_copy(maskv, cnt_sh.at[dstv], add=True)
            return ()

        lax.fori_loop(0, nch, body, ())
        plsc.subcore_barrier()
        off = 0
        while off < rps:
            c = min(EC, rps - off)
            pltpu.sync_copy(s_sh.at[pl.ds(r0 + off, c)], rows.at[pl.ds(0, c)])
            pltpu.sync_copy(rows.at[pl.ds(0, c)],
                            s_out.at[cid, pl.ds(r0 + off, c)])
            off += c
        pltpu.sync_copy(cnt_sh.at[pl.ds(r0, rps)], cvec.at[pl.ds(0, rps)])
        pltpu.sync_copy(cvec.at[pl.ds(0, rps)],
                        cnt_out.at[pl.ds(cid * n_pad + r0, rps)])

    s, cnt = agg(h, src, dst, mask, counts)
    return s[:, :n], cnt.reshape(NC, n_pad)[:, :n]


@functools.partial(jax.jit, static_argnums=(5,))
def _sc_gscatter(a, src, dst, mask, counts, n):
    e_pad = src.shape[0]
    epw = e_pad // NW
    rps = _rps(n)
    n_pad = rps * NS
    dummy = n
    mesh = plsc.VectorSubcoreMesh(core_axis_name="c", subcore_axis_name="s")

    @functools.partial(
        pl.kernel, mesh=mesh,
        out_type=jax.ShapeDtypeStruct((NC * n_pad,), jnp.float32),
        scratch_types=[
            pltpu.VMEM((EC,), jnp.int32),
            pltpu.VMEM((EC,), jnp.int32),
            pltpu.VMEM((EC,), jnp.float32),
            pltpu.VMEM((EC,), jnp.float32),
            pltpu.VMEM((-(-rps // 16) * 16,), jnp.float32),
            pltpu.VMEM_SHARED((n_pad,), jnp.float32),
            pltpu.VMEM((16,), jnp.int32),
            pltpu.SemaphoreType.DMA,
        ],
    )
    def gsc(a_hbm, src_hbm, dst_hbm, mask_hbm, counts_hbm,
            g_out, srcv, dstv, maskv, valsv, cvec, g_sh, cnts, sem):
        cid = lax.axis_index("c")
        sid = lax.axis_index("s")
        wid = sid * NC + cid
        r0 = sid * rps
        pltpu.sync_copy(counts_hbm.at[pl.ds(wid * 16, 16)], cnts)
        cv = cnts[...]
        nch = (cv[0] + (EC - 1)) // EC
        base = pl.multiple_of(cv[1], EC)
        for j in range(-(-rps // 16)):
            cvec[pl.ds(j * 16, 16)] = jnp.zeros((16,), jnp.float32)
        pltpu.sync_copy(cvec.at[pl.ds(0, rps)], g_sh.at[pl.ds(r0, rps)])
        plsc.subcore_barrier()

        def body(i, _):
            off = pl.multiple_of(base + i * EC, EC)
            pltpu.sync_copy(src_hbm.at[pl.ds(off, EC)], srcv)
            pltpu.sync_copy(dst_hbm.at[pl.ds(off, EC)], dstv)
            pltpu.sync_copy(mask_hbm.at[pl.ds(off, EC)], maskv)
            lane = lax.iota(jnp.int32, 16)
            for j in range(EC // 16):
                sl = pl.ds(j * 16, 16)
                m = maskv[sl]
                d = dstv[sl]
                dstv[sl] = jnp.where(m > 0.0, d, dummy + j * 16 + lane)
            pltpu.async_copy(a_hbm.at[srcv], valsv, sem).wait()
            pltpu.sync_copy(valsv, g_sh.at[dstv], add=True)
            return ()

        lax.fori_loop(0, nch, body, ())
        plsc.subcore_barrier()
        pltpu.sync_copy(g_sh.at[pl.ds(r0, rps)], cvec.at[pl.ds(0, rps)])
        pltpu.sync_copy(cvec.at[pl.ds(0, rps)],
                        g_out.at[pl.ds(cid * n_pad + r0, rps)])

    g = gsc(a, src, dst, mask, counts)
    return g.reshape(NC, n_pad)[:, :n]


def _sc_edgemap(src, dst, mask, counts, mapping):
    e_pad = src.shape[0]
    epw = e_pad // NW
    cpw = epw // EC
    mesh = plsc.VectorSubcoreMesh(core_axis_name="c", subcore_axis_name="s")

    @functools.partial(
        pl.kernel, mesh=mesh,
        out_type=[
            jax.ShapeDtypeStruct((e_pad,), jnp.int32),
            jax.ShapeDtypeStruct((e_pad,), jnp.int32),
            jax.ShapeDtypeStruct((e_pad,), jnp.float32),
        ],
        scratch_types=[
            pltpu.VMEM((EC,), jnp.int32),
            pltpu.VMEM((EC,), jnp.int32),
            pltpu.VMEM((EC,), jnp.int32),
            pltpu.VMEM((EC,), jnp.int32),
            pltpu.VMEM((EC,), jnp.float32),
            pltpu.VMEM((EC,), jnp.float32),
            pltpu.VMEM((16,), jnp.int32),
            pltpu.SemaphoreType.DMA,
        ],
    )
    def emap(src_hbm, dst_hbm, mask_hbm, counts_hbm, map_hbm,
             ms_out, md_out, nm_out,
             msv, mdv, srcv, dstv, maskv, nmv, cnts, sem):
        cid = lax.axis_index("c")
        sid = lax.axis_index("s")
        wid = sid * NC + cid
        pltpu.sync_copy(counts_hbm.at[pl.ds(wid * 16, 16)], cnts)
        cv = cnts[...]
        nch = (cv[0] + (EC - 1)) // EC
        base = pl.multiple_of(cv[1], EC)

        def body(i, _):
            off = pl.multiple_of(base + i * EC, EC)
            pltpu.sync_copy(src_hbm.at[pl.ds(off, EC)], srcv)
            pltpu.sync_copy(dst_hbm.at[pl.ds(off, EC)], dstv)
            pltpu.sync_copy(mask_hbm.at[pl.ds(off, EC)], maskv)
            pltpu.async_copy(map_hbm.at[srcv], msv, sem).wait()
            pltpu.async_copy(map_hbm.at[dstv], mdv, sem).wait()
            for j in range(EC // 16):
                sl = pl.ds(j * 16, 16)
                ms = msv[sl]
                md = mdv[sl]
                valid = (ms >= 0) & (md >= 0) & (maskv[sl] > 0.0)
                msv[sl] = jnp.where(valid, ms, 0)
                mdv[sl] = jnp.where(valid, md, 0)
                nmv[sl] = jnp.where(valid, 1.0, 0.0)
            pltpu.sync_copy(msv, ms_out.at[pl.ds(off, EC)])
            pltpu.sync_copy(mdv, md_out.at[pl.ds(off, EC)])
            pltpu.sync_copy(nmv, nm_out.at[pl.ds(off, EC)])
            return ()

        lax.fori_loop(0, nch, body, ())

    return emap(src, dst, mask, counts, mapping)


def _compact(ms, md, nm, prev_mask):
    e_pad = ms.shape[0]
    nmf = jnp.where(prev_mask > 0, nm, 0.0)
    cs = jnp.cumsum(nmf)
    total = cs[-1].astype(jnp.int32)
    pos = jnp.where(nmf > 0, cs.astype(jnp.int32) - 1, e_pad + 16)
    src2 = jnp.zeros((e_pad,), jnp.int32).at[pos].add(ms, mode="drop")
    dst2 = jnp.zeros((e_pad,), jnp.int32).at[pos].add(md, mode="drop")
    mask2 = (jnp.arange(e_pad) < total).astype(jnp.float32)
    w = jnp.arange(NW)
    starts = ((w * total) // NW) // EC * EC
    ends = jnp.concatenate([starts[1:], total[None]])
    cw = jnp.maximum(ends - starts, 0)
    counts2 = (jnp.zeros((NW, 16), jnp.int32)
               .at[:, 0].set(cw).at[:, 1].set(starts).reshape(-1))
    return src2, dst2, mask2, counts2


def _dense_body(s_ref, cnt_ref, h_ref, wl_ref, bl_ref, wr_ref, pw_ref, pb_ref,
                h1_ref, a_ref, dinv_ref, base_ref):
    s = s_ref[0] + s_ref[1]
    cnt = cnt_ref[0] + cnt_ref[1]
    h = h_ref[...]
    mean = s / jnp.maximum(cnt, 1.0)
    h1 = jnp.dot(mean, wl_ref[...], preferred_element_type=jnp.float32)
    h1 = h1 + bl_ref[...] + jnp.dot(h, wr_ref[...],
                                    preferred_element_type=jnp.float32)
    h1 = jnp.maximum(h1, 0.0)
    h1_ref[...] = h1
    xw = jnp.dot(h1, pw_ref[...], preferred_element_type=jnp.float32)
    deg = cnt + 1.0
    dinv = jax.lax.rsqrt(deg)
    a_ref[...] = xw * dinv
    dinv_ref[...] = dinv
    base_ref[...] = xw / deg + pb_ref[...]


def _dense_stage(s2, cnt2, h, Wl, bl, Wr, pW, pb):
    n = h.shape[0]
    R = 400
    grid = (n // R,)
    row = pl.BlockSpec((R, NHID), lambda i: (i, 0))
    prow = pl.BlockSpec((2, R, NHID), lambda i: (0, i, 0))
    pcol = pl.BlockSpec((2, R, 1), lambda i: (0, i, 0))
    col = pl.BlockSpec((R, 1), lambda i: (i, 0))
    full = pl.BlockSpec((NHID, NHID), lambda i: (0, 0))
    vec = pl.BlockSpec((1, NHID), lambda i: (0, 0))
    pws = pl.BlockSpec((NHID, 1), lambda i: (0, 0))
    pbs = pl.BlockSpec((1, 1), lambda i: (0, 0))
    h1, a, dinv, base = pl.pallas_call(
        _dense_body,
        grid=grid,
        in_specs=[prow, pcol, row, full, vec, full, pws, pbs],
        out_specs=[row, col, col, col],
        out_shape=[
            jax.ShapeDtypeStruct((n, NHID), jnp.float32),
            jax.ShapeDtypeStruct((n, 1), jnp.float32),
            jax.ShapeDtypeStruct((n, 1), jnp.float32),
            jax.ShapeDtypeStruct((n, 1), jnp.float32),
        ],
    )(s2, cnt2[..., None], h, Wl, bl.reshape(1, NHID), Wr, pW,
      pb.reshape(1, 1))
    return h1, a[:, 0], dinv[:, 0], base[:, 0]


def _head_body(x1_ref, x2_ref, x3_ref, w1_ref, b1_ref, w2_ref, b2_ref,
               w3_ref, b3_ref, feats_ref, out_ref):
    def readout(ref):
        v = ref[...]
        mx = jnp.max(v, axis=0, keepdims=True)
        mn = jnp.mean(v, axis=0, keepdims=True)
        return jnp.concatenate([mx, mn], axis=1)

    z = readout(x1_ref) + readout(x2_ref) + readout(x3_ref)
    z = jnp.dot(z, w1_ref[...], preferred_element_type=jnp.float32)
    z = jnp.maximum(z + b1_ref[...], 0.0)
    f = jnp.dot(z, w2_ref[...], preferred_element_type=jnp.float32)
    f = jnp.maximum(f + b2_ref[...], 0.0)
    feats_ref[...] = f
    out_ref[...] = jnp.dot(f, w3_ref[...],
                           preferred_element_type=jnp.float32) + b3_ref[...]


def _head(xk1, xk2, xk3, w1, b1, w2, b2, w3, b3):
    ncls = w3.shape[1]
    grph = w2.shape[1]
    feats, out = pl.pallas_call(
        _head_body,
        out_shape=[
            jax.ShapeDtypeStruct((1, grph), jnp.float32),
            jax.ShapeDtypeStruct((1, ncls), jnp.float32),
        ],
    )(xk1, xk2, xk3, w1, b1.reshape(1, -1), w2, b2.reshape(1, -1), w3,
      b3.reshape(1, -1))
    return feats, out


def _level(h, src, dst, mask, counts, Wl, bl, Wr, pW, pb, k, last=False):
    n = h.shape[0]
    s2, cnt2 = _sc_aggregate(h, src, dst, mask, counts, n)
    h1, a, dinv, base = _dense_stage(s2, cnt2, h, Wl, bl, Wr, pW, pb)
    g2 = _sc_gscatter(a, src, dst, mask, counts, n)
    score = jnp.tanh(dinv * (g2[0] + g2[1]) + base)
    top_scores, perm = jax.lax.top_k(score, k)
    xk = h1[perm] * top_scores[:, None]
    if last:
        return xk, None, None, None, None
    mapping = jnp.full((n,), -1, jnp.int32).at[perm].set(
        jnp.arange(k, dtype=jnp.int32))
    ms, md, nm = _sc_edgemap(src, dst, mask, counts, mapping)
    src2, dst2, mask2, counts2 = _compact(ms, md, nm, mask)
    return xk, src2, dst2, mask2, counts2


def kernel(x, edge_index, edge_attr, batch,
           conv1_Wl, conv1_bl, conv1_Wr, pool1_W, pool1_b,
           conv2_Wl, conv2_bl, conv2_Wr, pool2_W, pool2_b,
           conv3_Wl, conv3_bl, conv3_Wr, pool3_W, pool3_b,
           lin1_W, lin1_b, lin2_W, lin2_b, lin3_W, lin3_b):
    n = batch.shape[0]
    x = x[:n]
    e = edge_attr.shape[0]
    e_pad = -(-e // (NW * EC)) * (NW * EC)
    pad = e_pad - e
    epw = e_pad // NW
    src = jnp.concatenate([edge_index[0], jnp.zeros((pad,), jnp.int32)])
    dst = jnp.concatenate([edge_index[1], jnp.zeros((pad,), jnp.int32)])
    mask = jnp.concatenate([jnp.ones((e,), x.dtype),
                            jnp.zeros((pad,), x.dtype)])
    w = jnp.arange(NW)
    counts = (jnp.zeros((NW, 16), jnp.int32)
              .at[:, 0].set(epw).at[:, 1].set(w * epw).reshape(-1))
    k1 = int(math.ceil(RATIO * n))
    k2 = int(math.ceil(RATIO * k1))
    k3 = int(math.ceil(RATIO * k2))
    xk1, src, dst, mask, counts = _level(
        x, src, dst, mask, counts, conv1_Wl, conv1_bl, conv1_Wr,
        pool1_W, pool1_b, k1)
    xk2, src, dst, mask, counts = _level(
        xk1, src, dst, mask, counts, conv2_Wl, conv2_bl, conv2_Wr,
        pool2_W, pool2_b, k2)
    xk3, _, _, _, _ = _level(
        xk2, src, dst, mask, counts, conv3_Wl, conv3_bl, conv3_Wr,
        pool3_W, pool3_b, k3, last=True)
    return _head(xk1, xk2, xk3, lin1_W, lin1_b, lin2_W, lin2_b,
                 lin3_W, lin3_b)


# speedup vs baseline: 41.5206x; 1.1101x over previous
import functools
import math

import jax
import jax.numpy as jnp
from jax import lax
from jax.experimental import pallas as pl
from jax.experimental.pallas import tpu as pltpu
from jax.experimental.pallas import tpu_sc as plsc

NHID = 128
RATIO = 0.2
NC, NS, NW = 2, 16, 32
EC = 128
Q = 4
ECB = EC * Q
QA = 2
ECA = EC * QA


def _rps(n):
    r = -(-(n + EC) // NS)
    return -(-r // 8) * 8


@functools.partial(jax.jit, static_argnums=(5,))
def _sc_aggregate(h, src, dst, mask, counts, n):
    e_pad = src.shape[0]
    epw = e_pad // NW
    rps = _rps(n)
    n_pad = rps * NS
    dummy = n
    mesh = plsc.VectorSubcoreMesh(core_axis_name="c", subcore_axis_name="s")

    @functools.partial(
        pl.kernel, mesh=mesh,
        out_type=[
            jax.ShapeDtypeStruct((NC, n_pad, NHID), jnp.float32),
            jax.ShapeDtypeStruct((NC * n_pad,), jnp.float32),
        ],
        scratch_types=[
            pltpu.VMEM((ECA,), jnp.int32),
            pltpu.VMEM((QA, EC), jnp.int32),
            pltpu.VMEM((ECA,), jnp.float32),
            pltpu.VMEM((ECA, NHID), jnp.float32),
            pltpu.VMEM((-(-rps // 16) * 16,), jnp.float32),
            pltpu.VMEM_SHARED((n_pad, NHID), jnp.float32),
            pltpu.VMEM_SHARED((n_pad,), jnp.float32),
            pltpu.VMEM((16,), jnp.int32),
            pltpu.SemaphoreType.DMA,
        ],
    )
    def agg(h_hbm, src_hbm, dst_hbm, mask_hbm, counts_hbm,
            s_out, cnt_out, srcv, dstv2, maskv, rows, cvec, s_sh, cnt_sh,
            cnts, sem):
        cid = lax.axis_index("c")
        sid = lax.axis_index("s")
        wid = sid * NC + cid
        r0 = sid * rps
        pltpu.sync_copy(counts_hbm.at[pl.ds(wid * 16, 16)], cnts)
        cv = cnts[...]
        nch = (cv[0] + (ECA - 1)) // ECA
        base = pl.multiple_of(cv[1], EC)

        def zrow(r, _):
            for j in range(NHID // 16):
                rows[r, pl.ds(j * 16, 16)] = jnp.zeros((16,), jnp.float32)
            return ()
        lax.fori_loop(0, ECA, zrow, ())
        for j in range(-(-rps // 16)):
            cvec[pl.ds(j * 16, 16)] = jnp.zeros((16,), jnp.float32)
        off = 0
        while off < rps:
            c = min(ECA, rps - off)
            pltpu.sync_copy(rows.at[pl.ds(0, c)],
                            s_sh.at[pl.ds(r0 + off, c)])
            off += c
        pltpu.sync_copy(cvec.at[pl.ds(0, rps)], cnt_sh.at[pl.ds(r0, rps)])
        plsc.subcore_barrier()

        def body(i, _):
            off = pl.multiple_of(base + i * ECA, EC)
            hs = [pltpu.async_copy(src_hbm.at[pl.ds(off, ECA)], srcv, sem),
                  pltpu.async_copy(mask_hbm.at[pl.ds(off, ECA)], maskv, sem)]
            for q in range(QA):
                hs.append(pltpu.async_copy(
                    dst_hbm.at[pl.ds(off + q * EC, EC)], dstv2.at[q], sem))
            for hh in hs:
                hh.wait()
            lane = lax.iota(jnp.int32, 16)
            for q in range(QA):
                for j in range(EC // 16):
                    m = maskv[pl.ds(q * EC + j * 16, 16)]
                    d = dstv2[q, pl.ds(j * 16, 16)]
                    dstv2[q, pl.ds(j * 16, 16)] = jnp.where(
                        m > 0.0, d, dummy + j * 16 + lane)
            gs = [pltpu.async_copy(h_hbm.at[srcv.at[pl.ds(q * EC, EC)]],
                                   rows.at[pl.ds(q * EC, EC)], sem)
                  for q in range(QA)]
            for hh in gs:
                hh.wait()
            for q in range(QA):
                pltpu.sync_copy(rows.at[pl.ds(q * EC, EC)],
                                s_sh.at[dstv2.at[q]], add=True)
                pltpu.sync_copy(maskv.at[pl.ds(q * EC, EC)],
                                cnt_sh.at[dstv2.at[q]], add=True)
            return ()

        lax.fori_loop(0, nch, body, ())
        plsc.subcore_barrier()
        off = 0
        while off < rps:
            c = min(ECA, rps - off)
            pltpu.sync_copy(s_sh.at[pl.ds(r0 + off, c)], rows.at[pl.ds(0, c)])
            pltpu.sync_copy(rows.at[pl.ds(0, c)],
                            s_out.at[cid, pl.ds(r0 + off, c)])
            off += c
        pltpu.sync_copy(cnt_sh.at[pl.ds(r0, rps)], cvec.at[pl.ds(0, rps)])
        pltpu.sync_copy(cvec.at[pl.ds(0, rps)],
                        cnt_out.at[pl.ds(cid * n_pad + r0, rps)])

    s, cnt = agg(h, src, dst, mask, counts)
    return s[:, :n], cnt.reshape(NC, n_pad)[:, :n]


@functools.partial(jax.jit, static_argnums=(5,))
def _sc_gscatter(a, src, dst, mask, counts, n):
    e_pad = src.shape[0]
    epw = e_pad // NW
    rps = _rps(n)
    n_pad = rps * NS
    dummy = n
    mesh = plsc.VectorSubcoreMesh(core_axis_name="c", subcore_axis_name="s")

    @functools.partial(
        pl.kernel, mesh=mesh,
        out_type=jax.ShapeDtypeStruct((NC * n_pad,), jnp.float32),
        scratch_types=[
            pltpu.VMEM((ECB,), jnp.int32),
            pltpu.VMEM((Q, EC), jnp.int32),
            pltpu.VMEM((ECB,), jnp.float32),
            pltpu.VMEM((ECB,), jnp.float32),
            pltpu.VMEM((-(-rps // 16) * 16,), jnp.float32),
            pltpu.VMEM_SHARED((n_pad,), jnp.float32),
            pltpu.VMEM((16,), jnp.int32),
            pltpu.SemaphoreType.DMA,
        ],
    )
    def gsc(a_hbm, src_hbm, dst_hbm, mask_hbm, counts_hbm,
            g_out, srcv, dstv2, maskv, valsv, cvec, g_sh, cnts, sem):
        cid = lax.axis_index("c")
        sid = lax.axis_index("s")
        wid = sid * NC + cid
        r0 = sid * rps
        pltpu.sync_copy(counts_hbm.at[pl.ds(wid * 16, 16)], cnts)
        cv = cnts[...]
        nch = (cv[0] + (ECB - 1)) // ECB
        base = pl.multiple_of(cv[1], EC)
        for j in range(-(-rps // 16)):
            cvec[pl.ds(j * 16, 16)] = jnp.zeros((16,), jnp.float32)
        pltpu.sync_copy(cvec.at[pl.ds(0, rps)], g_sh.at[pl.ds(r0, rps)])
        plsc.subcore_barrier()

        def body(i, _):
            off = pl.multiple_of(base + i * ECB, EC)
            hs = [pltpu.async_copy(src_hbm.at[pl.ds(off, ECB)], srcv, sem),
                  pltpu.async_copy(mask_hbm.at[pl.ds(off, ECB)], maskv, sem)]
            for q in range(Q):
                hs.append(pltpu.async_copy(
                    dst_hbm.at[pl.ds(off + q * EC, EC)], dstv2.at[q], sem))
            for hh in hs:
                hh.wait()
            lane = lax.iota(jnp.int32, 16)
            for q in range(Q):
                for j in range(EC // 16):
                    m = maskv[pl.ds(q * EC + j * 16, 16)]
                    d = dstv2[q, pl.ds(j * 16, 16)]
                    dstv2[q, pl.ds(j * 16, 16)] = jnp.where(
                        m > 0.0, d, dummy + j * 16 + lane)
            gs = [pltpu.async_copy(a_hbm.at[srcv.at[pl.ds(q * EC, EC)]],
                                   valsv.at[pl.ds(q * EC, EC)], sem)
                  for q in range(Q)]
            for hh in gs:
                hh.wait()
            for q in range(Q):
                pltpu.sync_copy(valsv.at[pl.ds(q * EC, EC)],
                                g_sh.at[dstv2.at[q]], add=True)
            return ()

        lax.fori_loop(0, nch, body, ())
        plsc.subcore_barrier()
        pltpu.sync_copy(g_sh.at[pl.ds(r0, rps)], cvec.at[pl.ds(0, rps)])
        pltpu.sync_copy(cvec.at[pl.ds(0, rps)],
                        g_out.at[pl.ds(cid * n_pad + r0, rps)])

    g = gsc(a, src, dst, mask, counts)
    return g.reshape(NC, n_pad)[:, :n]


def _sc_edgemap(src, dst, mask, counts, mapping):
    e_pad = src.shape[0]
    epw = e_pad // NW
    mesh = plsc.VectorSubcoreMesh(core_axis_name="c", subcore_axis_name="s")

    @functools.partial(
        pl.kernel, mesh=mesh,
        out_type=[
            jax.ShapeDtypeStruct((e_pad,), jnp.int32),
            jax.ShapeDtypeStruct((e_pad,), jnp.int32),
            jax.ShapeDtypeStruct((e_pad,), jnp.float32),
        ],
        scratch_types=[
            pltpu.VMEM((ECB,), jnp.int32),
            pltpu.VMEM((ECB,), jnp.int32),
            pltpu.VMEM((ECB,), jnp.int32),
            pltpu.VMEM((ECB,), jnp.int32),
            pltpu.VMEM((ECB,), jnp.float32),
            pltpu.VMEM((ECB,), jnp.float32),
            pltpu.VMEM((16,), jnp.int32),
            pltpu.SemaphoreType.DMA,
        ],
    )
    def emap(src_hbm, dst_hbm, mask_hbm, counts_hbm, map_hbm,
             ms_out, md_out, nm_out,
             msv, mdv, srcv, dstv, maskv, nmv, cnts, sem):
        cid = lax.axis_index("c")
        sid = lax.axis_index("s")
        wid = sid * NC + cid
        pltpu.sync_copy(counts_hbm.at[pl.ds(wid * 16, 16)], cnts)
        cv = cnts[...]
        nch = (cv[0] + (ECB - 1)) // ECB
        base = pl.multiple_of(cv[1], EC)

        def body(i, _):
            off = pl.multiple_of(base + i * ECB, EC)
            hs = [pltpu.async_copy(src_hbm.at[pl.ds(off, ECB)], srcv, sem),
                  pltpu.async_copy(dst_hbm.at[pl.ds(off, ECB)], dstv, sem),
                  pltpu.async_copy(mask_hbm.at[pl.ds(off, ECB)], maskv, sem)]
            for hh in hs:
                hh.wait()
            gs = []
            for q in range(Q):
                sl = pl.ds(q * EC, EC)
                gs.append(pltpu.async_copy(map_hbm.at[srcv.at[sl]],
                                           msv.at[sl], sem))
                gs.append(pltpu.async_copy(map_hbm.at[dstv.at[sl]],
                                           mdv.at[sl], sem))
            for hh in gs:
                hh.wait()
            for j in range(ECB // 16):
                sl = pl.ds(j * 16, 16)
                ms = msv[sl]
                md = mdv[sl]
                valid = (ms >= 0) & (md >= 0) & (maskv[sl] > 0.0)
                msv[sl] = jnp.where(valid, ms, 0)
                mdv[sl] = jnp.where(valid, md, 0)
                nmv[sl] = jnp.where(valid, 1.0, 0.0)
            pltpu.sync_copy(msv, ms_out.at[pl.ds(off, ECB)])
            pltpu.sync_copy(mdv, md_out.at[pl.ds(off, ECB)])
            pltpu.sync_copy(nmv, nm_out.at[pl.ds(off, ECB)])
            return ()

        lax.fori_loop(0, nch, body, ())

    return emap(src, dst, mask, counts, mapping)


def _compact(ms, md, nm, prev_mask):
    e_pad = ms.shape[0]
    nmf = jnp.where(prev_mask > 0, nm, 0.0)
    cs = jnp.cumsum(nmf)
    total = cs[-1].astype(jnp.int32)
    pos = jnp.where(nmf > 0, cs.astype(jnp.int32) - 1, e_pad + 16)
    src2 = jnp.zeros((e_pad,), jnp.int32).at[pos].add(ms, mode="drop")
    dst2 = jnp.zeros((e_pad,), jnp.int32).at[pos].add(md, mode="drop")
    mask2 = (jnp.arange(e_pad) < total).astype(jnp.float32)
    w = jnp.arange(NW)
    starts = ((w * total) // NW) // ECB * ECB
    ends = jnp.concatenate([starts[1:], total[None]])
    cw = jnp.maximum(ends - starts, 0)
    counts2 = (jnp.zeros((NW, 16), jnp.int32)
               .at[:, 0].set(cw).at[:, 1].set(starts).reshape(-1))
    return src2, dst2, mask2, counts2


def _dense_body(s_ref, cnt_ref, h_ref, wl_ref, bl_ref, wr_ref, pw_ref, pb_ref,
                h1_ref, a_ref, dinv_ref, base_ref):
    s = s_ref[0] + s_ref[1]
    cnt = cnt_ref[0] + cnt_ref[1]
    h = h_ref[...]
    mean = s / jnp.maximum(cnt, 1.0)
    h1 = jnp.dot(mean, wl_ref[...], preferred_element_type=jnp.float32)
    h1 = h1 + bl_ref[...] + jnp.dot(h, wr_ref[...],
                                    preferred_element_type=jnp.float32)
    h1 = jnp.maximum(h1, 0.0)
    h1_ref[...] = h1
    xw = jnp.dot(h1, pw_ref[...], preferred_element_type=jnp.float32)
    deg = cnt + 1.0
    dinv = jax.lax.rsqrt(deg)
    a_ref[...] = xw * dinv
    dinv_ref[...] = dinv
    base_ref[...] = xw / deg + pb_ref[...]


def _dense_stage(s2, cnt2, h, Wl, bl, Wr, pW, pb):
    n = h.shape[0]
    R = 400
    grid = (n // R,)
    row = pl.BlockSpec((R, NHID), lambda i: (i, 0))
    prow = pl.BlockSpec((2, R, NHID), lambda i: (0, i, 0))
    pcol = pl.BlockSpec((2, R, 1), lambda i: (0, i, 0))
    col = pl.BlockSpec((R, 1), lambda i: (i, 0))
    full = pl.BlockSpec((NHID, NHID), lambda i: (0, 0))
    vec = pl.BlockSpec((1, NHID), lambda i: (0, 0))
    pws = pl.BlockSpec((NHID, 1), lambda i: (0, 0))
    pbs = pl.BlockSpec((1, 1), lambda i: (0, 0))
    h1, a, dinv, base = pl.pallas_call(
        _dense_body,
        grid=grid,
        in_specs=[prow, pcol, row, full, vec, full, pws, pbs],
        out_specs=[row, col, col, col],
        out_shape=[
            jax.ShapeDtypeStruct((n, NHID), jnp.float32),
            jax.ShapeDtypeStruct((n, 1), jnp.float32),
            jax.ShapeDtypeStruct((n, 1), jnp.float32),
            jax.ShapeDtypeStruct((n, 1), jnp.float32),
        ],
    )(s2, cnt2[..., None], h, Wl, bl.reshape(1, NHID), Wr, pW,
      pb.reshape(1, 1))
    return h1, a[:, 0], dinv[:, 0], base[:, 0]


def _head_body(x1_ref, x2_ref, x3_ref, w1_ref, b1_ref, w2_ref, b2_ref,
               w3_ref, b3_ref, feats_ref, out_ref):
    def readout(ref):
        v = ref[...]
        mx = jnp.max(v, axis=0, keepdims=True)
        mn = jnp.mean(v, axis=0, keepdims=True)
        return jnp.concatenate([mx, mn], axis=1)

    z = readout(x1_ref) + readout(x2_ref) + readout(x3_ref)
    z = jnp.dot(z, w1_ref[...], preferred_element_type=jnp.float32)
    z = jnp.maximum(z + b1_ref[...], 0.0)
    f = jnp.dot(z, w2_ref[...], preferred_element_type=jnp.float32)
    f = jnp.maximum(f + b2_ref[...], 0.0)
    feats_ref[...] = f
    out_ref[...] = jnp.dot(f, w3_ref[...],
                           preferred_element_type=jnp.float32) + b3_ref[...]


def _head(xk1, xk2, xk3, w1, b1, w2, b2, w3, b3):
    ncls = w3.shape[1]
    grph = w2.shape[1]
    feats, out = pl.pallas_call(
        _head_body,
        out_shape=[
            jax.ShapeDtypeStruct((1, grph), jnp.float32),
            jax.ShapeDtypeStruct((1, ncls), jnp.float32),
        ],
    )(xk1, xk2, xk3, w1, b1.reshape(1, -1), w2, b2.reshape(1, -1), w3,
      b3.reshape(1, -1))
    return feats, out


def _level(h, src, dst, mask, counts, Wl, bl, Wr, pW, pb, k, last=False):
    n = h.shape[0]
    s2, cnt2 = _sc_aggregate(h, src, dst, mask, counts, n)
    h1, a, dinv, base = _dense_stage(s2, cnt2, h, Wl, bl, Wr, pW, pb)
    g2 = _sc_gscatter(a, src, dst, mask, counts, n)
    score = jnp.tanh(dinv * (g2[0] + g2[1]) + base)
    top_scores, perm = jax.lax.top_k(score, k)
    xk = h1[perm] * top_scores[:, None]
    if last:
        return xk, None, None, None, None
    mapping = jnp.full((n,), -1, jnp.int32).at[perm].set(
        jnp.arange(k, dtype=jnp.int32))
    ms, md, nm = _sc_edgemap(src, dst, mask, counts, mapping)
    src2, dst2, mask2, counts2 = _compact(ms, md, nm, mask)
    return xk, src2, dst2, mask2, counts2


def kernel(x, edge_index, edge_attr, batch,
           conv1_Wl, conv1_bl, conv1_Wr, pool1_W, pool1_b,
           conv2_Wl, conv2_bl, conv2_Wr, pool2_W, pool2_b,
           conv3_Wl, conv3_bl, conv3_Wr, pool3_W, pool3_b,
           lin1_W, lin1_b, lin2_W, lin2_b, lin3_W, lin3_b):
    n = batch.shape[0]
    x = x[:n]
    e = edge_attr.shape[0]
    e_pad = -(-e // (NW * ECB)) * (NW * ECB)
    pad = e_pad - e
    epw = e_pad // NW
    src = jnp.concatenate([edge_index[0], jnp.zeros((pad,), jnp.int32)])
    dst = jnp.concatenate([edge_index[1], jnp.zeros((pad,), jnp.int32)])
    mask = jnp.concatenate([jnp.ones((e,), x.dtype),
                            jnp.zeros((pad,), x.dtype)])
    w = jnp.arange(NW)
    counts = (jnp.zeros((NW, 16), jnp.int32)
              .at[:, 0].set(epw).at[:, 1].set(w * epw).reshape(-1))
    k1 = int(math.ceil(RATIO * n))
    k2 = int(math.ceil(RATIO * k1))
    k3 = int(math.ceil(RATIO * k2))
    xk1, src, dst, mask, counts = _level(
        x, src, dst, mask, counts, conv1_Wl, conv1_bl, conv1_Wr,
        pool1_W, pool1_b, k1)
    xk2, src, dst, mask, counts = _level(
        xk1, src, dst, mask, counts, conv2_Wl, conv2_bl, conv2_Wr,
        pool2_W, pool2_b, k2)
    xk3, _, _, _, _ = _level(
        xk2, src, dst, mask, counts, conv3_Wl, conv3_bl, conv3_Wr,
        pool3_W, pool3_b, k3, last=True)
    return _head(xk1, xk2, xk3, lin1_W, lin1_b, lin2_W, lin2_b,
                 lin3_W, lin3_b)


# speedup vs baseline: 42.6770x; 1.0279x over previous
import functools
import math

import jax
import jax.numpy as jnp
from jax import lax
from jax.experimental import pallas as pl
from jax.experimental.pallas import tpu as pltpu
from jax.experimental.pallas import tpu_sc as plsc

NHID = 128
RATIO = 0.2
NC, NS, NW = 2, 16, 32
EC = 128
Q = 4
ECB = EC * Q
QA = 2
ECA = EC * QA


def _rps(n):
    r = -(-(n + EC) // NS)
    return -(-r // 8) * 8


@functools.partial(jax.jit, static_argnums=(5,))
def _sc_aggregate(h, src, dst, mask, counts, n):
    e_pad = src.shape[0]
    epw = e_pad // NW
    rps = _rps(n)
    n_pad = rps * NS
    dummy = n
    mesh = plsc.VectorSubcoreMesh(core_axis_name="c", subcore_axis_name="s")

    @functools.partial(
        pl.kernel, mesh=mesh,
        out_type=[
            jax.ShapeDtypeStruct((NC, n_pad, NHID), jnp.float32),
            jax.ShapeDtypeStruct((NC * n_pad,), jnp.float32),
        ],
        scratch_types=[
            pltpu.VMEM((ECA,), jnp.int32),
            pltpu.VMEM((QA, EC), jnp.int32),
            pltpu.VMEM((ECA,), jnp.float32),
            pltpu.VMEM((ECA, NHID), jnp.float32),
            pltpu.VMEM((-(-rps // 16) * 16,), jnp.float32),
            pltpu.VMEM_SHARED((n_pad, NHID), jnp.float32),
            pltpu.VMEM_SHARED((n_pad,), jnp.float32),
            pltpu.VMEM((16,), jnp.int32),
            pltpu.SemaphoreType.DMA,
            pltpu.SemaphoreType.DMA,
            pltpu.SemaphoreType.DMA,
        ],
    )
    def agg(h_hbm, src_hbm, dst_hbm, mask_hbm, counts_hbm,
            s_out, cnt_out, srcv, dstv2, maskv, rows, cvec, s_sh, cnt_sh,
            cnts, sem, semi, semg):
        cid = lax.axis_index("c")
        sid = lax.axis_index("s")
        wid = sid * NC + cid
        r0 = sid * rps
        pltpu.sync_copy(counts_hbm.at[pl.ds(wid * 16, 16)], cnts)
        cv = cnts[...]
        nch = (cv[0] + (ECA - 1)) // ECA
        base = pl.multiple_of(cv[1], EC)

        def zrow(r, _):
            for j in range(NHID // 16):
                rows[r, pl.ds(j * 16, 16)] = jnp.zeros((16,), jnp.float32)
            return ()
        lax.fori_loop(0, ECA, zrow, ())
        for j in range(-(-rps // 16)):
            cvec[pl.ds(j * 16, 16)] = jnp.zeros((16,), jnp.float32)
        off = 0
        while off < rps:
            c = min(ECA, rps - off)
            pltpu.sync_copy(rows.at[pl.ds(0, c)],
                            s_sh.at[pl.ds(r0 + off, c)])
            off += c
        pltpu.sync_copy(cvec.at[pl.ds(0, rps)], cnt_sh.at[pl.ds(r0, rps)])
        plsc.subcore_barrier()

        def body(i, _):
            off = pl.multiple_of(base + i * ECA, EC)
            hsrc = pltpu.async_copy(src_hbm.at[pl.ds(off, ECA)], srcv, sem)
            hs = [pltpu.async_copy(mask_hbm.at[pl.ds(off, ECA)], maskv, semi)]
            for q in range(QA):
                hs.append(pltpu.async_copy(
                    dst_hbm.at[pl.ds(off + q * EC, EC)], dstv2.at[q], semi))
            hsrc.wait()
            gs = [pltpu.async_copy(h_hbm.at[srcv.at[pl.ds(q * EC, EC)]],
                                   rows.at[pl.ds(q * EC, EC)], semg)
                  for q in range(QA)]
            for hh in hs:
                hh.wait()
            lane = lax.iota(jnp.int32, 16)
            for q in range(QA):
                for j in range(EC // 16):
                    m = maskv[pl.ds(q * EC + j * 16, 16)]
                    d = dstv2[q, pl.ds(j * 16, 16)]
                    dstv2[q, pl.ds(j * 16, 16)] = jnp.where(
                        m > 0.0, d, dummy + j * 16 + lane)
            for q in range(QA):
                gs[q].wait()
                pltpu.sync_copy(rows.at[pl.ds(q * EC, EC)],
                                s_sh.at[dstv2.at[q]], add=True)
                pltpu.sync_copy(maskv.at[pl.ds(q * EC, EC)],
                                cnt_sh.at[dstv2.at[q]], add=True)
            return ()

        lax.fori_loop(0, nch, body, ())
        plsc.subcore_barrier()
        off = 0
        while off < rps:
            c = min(ECA, rps - off)
            pltpu.sync_copy(s_sh.at[pl.ds(r0 + off, c)], rows.at[pl.ds(0, c)])
            pltpu.sync_copy(rows.at[pl.ds(0, c)],
                            s_out.at[cid, pl.ds(r0 + off, c)])
            off += c
        pltpu.sync_copy(cnt_sh.at[pl.ds(r0, rps)], cvec.at[pl.ds(0, rps)])
        pltpu.sync_copy(cvec.at[pl.ds(0, rps)],
                        cnt_out.at[pl.ds(cid * n_pad + r0, rps)])

    s, cnt = agg(h, src, dst, mask, counts)
    return s[:, :n], cnt.reshape(NC, n_pad)[:, :n]


@functools.partial(jax.jit, static_argnums=(5,))
def _sc_gscatter(a, src, dst, mask, counts, n):
    e_pad = src.shape[0]
    epw = e_pad // NW
    rps = _rps(n)
    n_pad = rps * NS
    dummy = n
    mesh = plsc.VectorSubcoreMesh(core_axis_name="c", subcore_axis_name="s")

    @functools.partial(
        pl.kernel, mesh=mesh,
        out_type=jax.ShapeDtypeStruct((NC * n_pad,), jnp.float32),
        scratch_types=[
            pltpu.VMEM((ECB,), jnp.int32),
            pltpu.VMEM((Q, EC), jnp.int32),
            pltpu.VMEM((ECB,), jnp.float32),
            pltpu.VMEM((ECB,), jnp.float32),
            pltpu.VMEM((-(-rps // 16) * 16,), jnp.float32),
            pltpu.VMEM_SHARED((n_pad,), jnp.float32),
            pltpu.VMEM((16,), jnp.int32),
            pltpu.SemaphoreType.DMA,
            pltpu.SemaphoreType.DMA,
            pltpu.SemaphoreType.DMA,
        ],
    )
    def gsc(a_hbm, src_hbm, dst_hbm, mask_hbm, counts_hbm,
            g_out, srcv, dstv2, maskv, valsv, cvec, g_sh, cnts,
            sem, semi, semg):
        cid = lax.axis_index("c")
        sid = lax.axis_index("s")
        wid = sid * NC + cid
        r0 = sid * rps
        pltpu.sync_copy(counts_hbm.at[pl.ds(wid * 16, 16)], cnts)
        cv = cnts[...]
        nch = (cv[0] + (ECB - 1)) // ECB
        base = pl.multiple_of(cv[1], EC)
        for j in range(-(-rps // 16)):
            cvec[pl.ds(j * 16, 16)] = jnp.zeros((16,), jnp.float32)
        pltpu.sync_copy(cvec.at[pl.ds(0, rps)], g_sh.at[pl.ds(r0, rps)])
        plsc.subcore_barrier()

        def body(i, _):
            off = pl.multiple_of(base + i * ECB, EC)
            hsrc = pltpu.async_copy(src_hbm.at[pl.ds(off, ECB)], srcv, sem)
            hs = [pltpu.async_copy(mask_hbm.at[pl.ds(off, ECB)], maskv, semi)]
            for q in range(Q):
                hs.append(pltpu.async_copy(
                    dst_hbm.at[pl.ds(off + q * EC, EC)], dstv2.at[q], semi))
            hsrc.wait()
            gs = [pltpu.async_copy(a_hbm.at[srcv.at[pl.ds(q * EC, EC)]],
                                   valsv.at[pl.ds(q * EC, EC)], semg)
                  for q in range(Q)]
            for hh in hs:
                hh.wait()
            lane = lax.iota(jnp.int32, 16)
            for q in range(Q):
                for j in range(EC // 16):
                    m = maskv[pl.ds(q * EC + j * 16, 16)]
                    d = dstv2[q, pl.ds(j * 16, 16)]
                    dstv2[q, pl.ds(j * 16, 16)] = jnp.where(
                        m > 0.0, d, dummy + j * 16 + lane)
            for q in range(Q):
                gs[q].wait()
                pltpu.sync_copy(valsv.at[pl.ds(q * EC, EC)],
                                g_sh.at[dstv2.at[q]], add=True)
            return ()

        lax.fori_loop(0, nch, body, ())
        plsc.subcore_barrier()
        pltpu.sync_copy(g_sh.at[pl.ds(r0, rps)], cvec.at[pl.ds(0, rps)])
        pltpu.sync_copy(cvec.at[pl.ds(0, rps)],
                        g_out.at[pl.ds(cid * n_pad + r0, rps)])

    g = gsc(a, src, dst, mask, counts)
    return g.reshape(NC, n_pad)[:, :n]


def _sc_edgemap(src, dst, mask, counts, mapping):
    e_pad = src.shape[0]
    epw = e_pad // NW
    mesh = plsc.VectorSubcoreMesh(core_axis_name="c", subcore_axis_name="s")

    @functools.partial(
        pl.kernel, mesh=mesh,
        out_type=[
            jax.ShapeDtypeStruct((e_pad,), jnp.int32),
            jax.ShapeDtypeStruct((e_pad,), jnp.int32),
            jax.ShapeDtypeStruct((e_pad,), jnp.float32),
        ],
        scratch_types=[
            pltpu.VMEM((ECB,), jnp.int32),
            pltpu.VMEM((ECB,), jnp.int32),
            pltpu.VMEM((ECB,), jnp.int32),
            pltpu.VMEM((ECB,), jnp.int32),
            pltpu.VMEM((ECB,), jnp.float32),
            pltpu.VMEM((ECB,), jnp.float32),
            pltpu.VMEM((16,), jnp.int32),
            pltpu.SemaphoreType.DMA,
        ],
    )
    def emap(src_hbm, dst_hbm, mask_hbm, counts_hbm, map_hbm,
             ms_out, md_out, nm_out,
             msv, mdv, srcv, dstv, maskv, nmv, cnts, sem):
        cid = lax.axis_index("c")
        sid = lax.axis_index("s")
        wid = sid * NC + cid
        pltpu.sync_copy(counts_hbm.at[pl.ds(wid * 16, 16)], cnts)
        cv = cnts[...]
        nch = (cv[0] + (ECB - 1)) // ECB
        base = pl.multiple_of(cv[1], EC)

        def body(i, _):
            off = pl.multiple_of(base + i * ECB, EC)
            hs = [pltpu.async_copy(src_hbm.at[pl.ds(off, ECB)], srcv, sem),
                  pltpu.async_copy(dst_hbm.at[pl.ds(off, ECB)], dstv, sem),
                  pltpu.async_copy(mask_hbm.at[pl.ds(off, ECB)], maskv, sem)]
            for hh in hs:
                hh.wait()
            gs = []
            for q in range(Q):
                sl = pl.ds(q * EC, EC)
                gs.append(pltpu.async_copy(map_hbm.at[srcv.at[sl]],
                                           msv.at[sl], sem))
                gs.append(pltpu.async_copy(map_hbm.at[dstv.at[sl]],
                                           mdv.at[sl], sem))
            for hh in gs:
                hh.wait()
            for j in range(ECB // 16):
                sl = pl.ds(j * 16, 16)
                ms = msv[sl]
                md = mdv[sl]
                valid = (ms >= 0) & (md >= 0) & (maskv[sl] > 0.0)
                msv[sl] = jnp.where(valid, ms, 0)
                mdv[sl] = jnp.where(valid, md, 0)
                nmv[sl] = jnp.where(valid, 1.0, 0.0)
            pltpu.sync_copy(msv, ms_out.at[pl.ds(off, ECB)])
            pltpu.sync_copy(mdv, md_out.at[pl.ds(off, ECB)])
            pltpu.sync_copy(nmv, nm_out.at[pl.ds(off, ECB)])
            return ()

        lax.fori_loop(0, nch, body, ())

    return emap(src, dst, mask, counts, mapping)


def _compact(ms, md, nm, prev_mask):
    e_pad = ms.shape[0]
    nmf = jnp.where(prev_mask > 0, nm, 0.0)
    cs = jnp.cumsum(nmf)
    total = cs[-1].astype(jnp.int32)
    pos = jnp.where(nmf > 0, cs.astype(jnp.int32) - 1, e_pad + 16)
    src2 = jnp.zeros((e_pad,), jnp.int32).at[pos].add(ms, mode="drop")
    dst2 = jnp.zeros((e_pad,), jnp.int32).at[pos].add(md, mode="drop")
    mask2 = (jnp.arange(e_pad) < total).astype(jnp.float32)
    w = jnp.arange(NW)
    starts = ((w * total) // NW) // ECB * ECB
    ends = jnp.concatenate([starts[1:], total[None]])
    cw = jnp.maximum(ends - starts, 0)
    counts2 = (jnp.zeros((NW, 16), jnp.int32)
               .at[:, 0].set(cw).at[:, 1].set(starts).reshape(-1))
    return src2, dst2, mask2, counts2


def _dense_body(s_ref, cnt_ref, h_ref, wl_ref, bl_ref, wr_ref, pw_ref, pb_ref,
                h1_ref, a_ref, dinv_ref, base_ref):
    s = s_ref[0] + s_ref[1]
    cnt = cnt_ref[0] + cnt_ref[1]
    h = h_ref[...]
    mean = s / jnp.maximum(cnt, 1.0)
    h1 = jnp.dot(mean, wl_ref[...], preferred_element_type=jnp.float32)
    h1 = h1 + bl_ref[...] + jnp.dot(h, wr_ref[...],
                                    preferred_element_type=jnp.float32)
    h1 = jnp.maximum(h1, 0.0)
    h1_ref[...] = h1
    xw = jnp.dot(h1, pw_ref[...], preferred_element_type=jnp.float32)
    deg = cnt + 1.0
    dinv = jax.lax.rsqrt(deg)
    a_ref[...] = xw * dinv
    dinv_ref[...] = dinv
    base_ref[...] = xw / deg + pb_ref[...]


def _dense_stage(s2, cnt2, h, Wl, bl, Wr, pW, pb):
    n = h.shape[0]
    R = 400
    grid = (n // R,)
    row = pl.BlockSpec((R, NHID), lambda i: (i, 0))
    prow = pl.BlockSpec((2, R, NHID), lambda i: (0, i, 0))
    pcol = pl.BlockSpec((2, R, 1), lambda i: (0, i, 0))
    col = pl.BlockSpec((R, 1), lambda i: (i, 0))
    full = pl.BlockSpec((NHID, NHID), lambda i: (0, 0))
    vec = pl.BlockSpec((1, NHID), lambda i: (0, 0))
    pws = pl.BlockSpec((NHID, 1), lambda i: (0, 0))
    pbs = pl.BlockSpec((1, 1), lambda i: (0, 0))
    h1, a, dinv, base = pl.pallas_call(
        _dense_body,
        grid=grid,
        in_specs=[prow, pcol, row, full, vec, full, pws, pbs],
        out_specs=[row, col, col, col],
        out_shape=[
            jax.ShapeDtypeStruct((n, NHID), jnp.float32),
            jax.ShapeDtypeStruct((n, 1), jnp.float32),
            jax.ShapeDtypeStruct((n, 1), jnp.float32),
            jax.ShapeDtypeStruct((n, 1), jnp.float32),
        ],
    )(s2, cnt2[..., None], h, Wl, bl.reshape(1, NHID), Wr, pW,
      pb.reshape(1, 1))
    return h1, a[:, 0], dinv[:, 0], base[:, 0]


def _head_body(x1_ref, x2_ref, x3_ref, w1_ref, b1_ref, w2_ref, b2_ref,
               w3_ref, b3_ref, feats_ref, out_ref):
    def readout(ref):
        v = ref[...]
        mx = jnp.max(v, axis=0, keepdims=True)
        mn = jnp.mean(v, axis=0, keepdims=True)
        return jnp.concatenate([mx, mn], axis=1)

    z = readout(x1_ref) + readout(x2_ref) + readout(x3_ref)
    z = jnp.dot(z, w1_ref[...], preferred_element_type=jnp.float32)
    z = jnp.maximum(z + b1_ref[...], 0.0)
    f = jnp.dot(z, w2_ref[...], preferred_element_type=jnp.float32)
    f = jnp.maximum(f + b2_ref[...], 0.0)
    feats_ref[...] = f
    out_ref[...] = jnp.dot(f, w3_ref[...],
                           preferred_element_type=jnp.float32) + b3_ref[...]


def _head(xk1, xk2, xk3, w1, b1, w2, b2, w3, b3):
    ncls = w3.shape[1]
    grph = w2.shape[1]
    feats, out = pl.pallas_call(
        _head_body,
        out_shape=[
            jax.ShapeDtypeStruct((1, grph), jnp.float32),
            jax.ShapeDtypeStruct((1, ncls), jnp.float32),
        ],
    )(xk1, xk2, xk3, w1, b1.reshape(1, -1), w2, b2.reshape(1, -1), w3,
      b3.reshape(1, -1))
    return feats, out


def _level(h, src, dst, mask, counts, Wl, bl, Wr, pW, pb, k, last=False):
    n = h.shape[0]
    s2, cnt2 = _sc_aggregate(h, src, dst, mask, counts, n)
    h1, a, dinv, base = _dense_stage(s2, cnt2, h, Wl, bl, Wr, pW, pb)
    g2 = _sc_gscatter(a, src, dst, mask, counts, n)
    score = jnp.tanh(dinv * (g2[0] + g2[1]) + base)
    top_scores, perm = jax.lax.top_k(score, k)
    xk = h1[perm] * top_scores[:, None]
    if last:
        return xk, None, None, None, None
    mapping = jnp.full((n,), -1, jnp.int32).at[perm].set(
        jnp.arange(k, dtype=jnp.int32))
    ms, md, nm = _sc_edgemap(src, dst, mask, counts, mapping)
    src2, dst2, mask2, counts2 = _compact(ms, md, nm, mask)
    return xk, src2, dst2, mask2, counts2


def kernel(x, edge_index, edge_attr, batch,
           conv1_Wl, conv1_bl, conv1_Wr, pool1_W, pool1_b,
           conv2_Wl, conv2_bl, conv2_Wr, pool2_W, pool2_b,
           conv3_Wl, conv3_bl, conv3_Wr, pool3_W, pool3_b,
           lin1_W, lin1_b, lin2_W, lin2_b, lin3_W, lin3_b):
    n = batch.shape[0]
    x = x[:n]
    e = edge_attr.shape[0]
    e_pad = -(-e // (NW * ECB)) * (NW * ECB)
    pad = e_pad - e
    epw = e_pad // NW
    src = jnp.concatenate([edge_index[0], jnp.zeros((pad,), jnp.int32)])
    dst = jnp.concatenate([edge_index[1], jnp.zeros((pad,), jnp.int32)])
    mask = jnp.concatenate([jnp.ones((e,), x.dtype),
                            jnp.zeros((pad,), x.dtype)])
    w = jnp.arange(NW)
    counts = (jnp.zeros((NW, 16), jnp.int32)
              .at[:, 0].set(epw).at[:, 1].set(w * epw).reshape(-1))
    k1 = int(math.ceil(RATIO * n))
    k2 = int(math.ceil(RATIO * k1))
    k3 = int(math.ceil(RATIO * k2))
    xk1, src, dst, mask, counts = _level(
        x, src, dst, mask, counts, conv1_Wl, conv1_bl, conv1_Wr,
        pool1_W, pool1_b, k1)
    xk2, src, dst, mask, counts = _level(
        xk1, src, dst, mask, counts, conv2_Wl, conv2_bl, conv2_Wr,
        pool2_W, pool2_b, k2)
    xk3, _, _, _, _ = _level(
        xk2, src, dst, mask, counts, conv3_Wl, conv3_bl, conv3_Wr,
        pool3_W, pool3_b, k3, last=True)
    return _head(xk1, xk2, xk3, lin1_W, lin1_b, lin2_W, lin2_b,
                 lin3_W, lin3_b)
